# baseline (speedup 1.0000x reference)
import math

import jax
import jax.numpy as jnp
from jax.experimental import pallas as pl
from jax.experimental.pallas import tpu as pltpu

B, S, D = 2, 2048, 1024
N, HID, HD, G = 16, 64, 64, 64
T = B * S
TM = 1024
PACK = 4

_ERF_C = (0.7978844936834865, -0.13297926880591132, 0.019941680140661174,
          -0.0023670825424680093, 0.00022562852359564684,
          -1.6874089254329677e-05, 8.80958264538878e-07,
          -2.327365050680541e-08)


def _gelu(h):
    s = h * h
    q = jnp.float32(_ERF_C[-1])
    for c in _ERF_C[-2::-1]:
        q = q * s + jnp.float32(c)
    return 0.5 * h * (1.0 + h * q)


def _body(temp_ref, x_ref, rwt_ref, ct_ref, win2_ref, bd4_ref, out_ref):
    xb = x_ref[:]

    z = jnp.dot(xb, rwt_ref[:], preferred_element_type=jnp.float32)
    zn = z / jnp.maximum(jnp.sqrt(jnp.sum(z * z, axis=1, keepdims=True)), 1e-12)
    ct = ct_ref[:]
    cn = ct / jnp.maximum(jnp.sqrt(jnp.sum(ct * ct, axis=0, keepdims=True)), 1e-12)
    logits = jnp.dot(zn.astype(jnp.bfloat16), cn.astype(jnp.bfloat16),
                     preferred_element_type=jnp.float32) / math.sqrt(G)
    logits = logits * jnp.exp(temp_ref[0, 0])

    nidx = jax.lax.broadcasted_iota(jnp.int32, (TM, N), 1)
    neg = jnp.float32(-1e30)
    m1 = jnp.max(logits, axis=1, keepdims=True)
    i1 = jnp.min(jnp.where(logits == m1, nidx, N), axis=1, keepdims=True)
    l2 = jnp.where(nidx == i1, neg, logits)
    m2 = jnp.max(l2, axis=1, keepdims=True)
    i2 = jnp.min(jnp.where(l2 == m2, nidx, N), axis=1, keepdims=True)

    e = jnp.exp(logits - m1)
    probs = e / jnp.sum(e, axis=1, keepdims=True)
    p1 = jnp.max(probs, axis=1, keepdims=True)
    p2 = jnp.max(jnp.where(nidx == i1, neg, probs), axis=1, keepdims=True)
    w = jnp.where(nidx == i1, p1, 0.0) + jnp.where(nidx == i2, p2, 0.0)
    wexp = jnp.repeat(w, HD, axis=1)

    hidden = jnp.dot(xb, win2_ref[:], preferred_element_type=jnp.float32)
    hb = _gelu(hidden).astype(jnp.bfloat16)
    kp = PACK * HID
    for g in range(N // PACK):
        out_ref[:, g * kp:(g + 1) * kp] = jnp.dot(
            hb[:, g * kp:(g + 1) * kp], bd4_ref[g],
            preferred_element_type=jnp.float32) * wexp[:, g * kp:(g + 1) * kp]


def kernel(x, router_w, head_centroids, temperature, input_experts, output_experts):
    xb = x.reshape(T, D).astype(jnp.bfloat16)
    rwt = router_w.T.astype(jnp.bfloat16)
    ct = head_centroids.T
    win2 = input_experts.transpose(1, 0, 2).reshape(D, N * HID).astype(jnp.bfloat16)
    w4 = output_experts.reshape(N // PACK, PACK, HID, HD)
    eye = jnp.eye(PACK, dtype=jnp.float32)
    bd4 = jnp.einsum('gadh,ab->gadbh', w4, eye).reshape(
        N // PACK, PACK * HID, PACK * HD).astype(jnp.bfloat16)
    temp = temperature.reshape(1, 1)

    out = pl.pallas_call(
        _body,
        grid=(T // TM,),
        in_specs=[
            pl.BlockSpec(memory_space=pltpu.SMEM),
            pl.BlockSpec((TM, D), lambda i: (i, 0)),
            pl.BlockSpec((D, G), lambda i: (0, 0)),
            pl.BlockSpec((G, N), lambda i: (0, 0)),
            pl.BlockSpec((D, N * HID), lambda i: (0, 0)),
            pl.BlockSpec((N // PACK, PACK * HID, PACK * HD), lambda i: (0, 0, 0)),
        ],
        out_specs=pl.BlockSpec((TM, N * HD), lambda i: (i, 0)),
        out_shape=jax.ShapeDtypeStruct((T, N * HD), jnp.float32),
        compiler_params=pltpu.CompilerParams(dimension_semantics=("arbitrary",)),
    )(temp, xb, rwt, ct, win2, bd4)
    return out.reshape(B, S, N * HD)


# speedup vs baseline: 1.4089x; 1.4089x over previous
import math

import jax
import jax.numpy as jnp
from jax import lax
from jax.experimental import pallas as pl
from jax.experimental.pallas import tpu as pltpu

B, S, D = 2, 2048, 1024
N, HID, HD, G = 16, 64, 64, 64
T = B * S
TM = 1024
PACK = 4
KP = PACK * HID

_ERF_C = (0.7978757940706152, -0.13286748344154034, 0.01970914059090927,
          -0.0021886570044185915, 0.0001622154401614631,
          -5.822014968681621e-06)

_TT = (((1,), (1,)), ((), ()))


def _gelu(h):
    s = h * h
    q = jnp.float32(_ERF_C[-1])
    for c in _ERF_C[-2::-1]:
        q = q * s + jnp.float32(c)
    return 0.5 * h * (1.0 + h * q)


def _body(temp_ref, x_ref, rw_ref, c_ref, win_ref, wout_ref, out_ref,
          win2_s, bd_s, emat_s):
    @pl.when(pl.program_id(0) == 0)
    def _prep():
        for n in range(N):
            win2_s[:, n * HID:(n + 1) * HID] = win_ref[n].astype(jnp.bfloat16)
        bd_s[:] = jnp.zeros((N // PACK, KP, KP), jnp.bfloat16)
        for g in range(N // PACK):
            for a in range(PACK):
                bd_s[g, a * HID:(a + 1) * HID, a * HD:(a + 1) * HD] = (
                    wout_ref[g * PACK + a].astype(jnp.bfloat16))
        lane = lax.broadcasted_iota(jnp.int32, (N, N * HD), 1) // HD
        sub = lax.broadcasted_iota(jnp.int32, (N, N * HD), 0)
        emat_s[:] = (lane == sub).astype(jnp.bfloat16)

    xb = x_ref[:].astype(jnp.bfloat16)

    z = lax.dot_general(xb, rw_ref[:].astype(jnp.bfloat16), _TT,
                        preferred_element_type=jnp.float32)
    zn = z / jnp.maximum(jnp.sqrt(jnp.sum(z * z, axis=1, keepdims=True)), 1e-12)
    c = c_ref[:]
    cn = c / jnp.maximum(jnp.sqrt(jnp.sum(c * c, axis=1, keepdims=True)), 1e-12)
    logits = lax.dot_general(zn.astype(jnp.bfloat16), cn.astype(jnp.bfloat16),
                             _TT, preferred_element_type=jnp.float32)
    logits = logits * (jnp.exp(temp_ref[0, 0]) / math.sqrt(G))

    nidx = lax.broadcasted_iota(jnp.int32, (TM, N), 1)
    neg = jnp.float32(-1e30)
    m1 = jnp.max(logits, axis=1, keepdims=True)
    i1 = jnp.min(jnp.where(logits == m1, nidx, N), axis=1, keepdims=True)
    l2 = jnp.where(nidx == i1, neg, logits)
    m2 = jnp.max(l2, axis=1, keepdims=True)
    i2 = jnp.min(jnp.where(l2 == m2, nidx, N), axis=1, keepdims=True)

    e = jnp.exp(logits - m1)
    probs = e / jnp.sum(e, axis=1, keepdims=True)
    p1 = jnp.max(probs, axis=1, keepdims=True)
    p2 = jnp.max(jnp.where(nidx == i1, neg, probs), axis=1, keepdims=True)
    w = jnp.where(nidx == i1, p1, 0.0) + jnp.where(nidx == i2, p2, 0.0)
    wexp = jnp.dot(w.astype(jnp.bfloat16), emat_s[:],
                   preferred_element_type=jnp.float32)

    hidden = jnp.dot(xb, win2_s[:], preferred_element_type=jnp.float32)
    hb = _gelu(hidden).astype(jnp.bfloat16)
    for g in range(N // PACK):
        out_ref[:, g * KP:(g + 1) * KP] = lax.dot_general(
            hb[:, g * KP:(g + 1) * KP], bd_s[g], (((1,), (0,)), ((), ())),
            preferred_element_type=jnp.float32) * wexp[:, g * KP:(g + 1) * KP]


def kernel(x, router_w, head_centroids, temperature, input_experts, output_experts):
    out = pl.pallas_call(
        _body,
        grid=(T // TM,),
        in_specs=[
            pl.BlockSpec(memory_space=pltpu.SMEM),
            pl.BlockSpec((TM, D), lambda i: (i, 0)),
            pl.BlockSpec((G, D), lambda i: (0, 0)),
            pl.BlockSpec((N, G), lambda i: (0, 0)),
            pl.BlockSpec((N, D, HID), lambda i: (0, 0, 0)),
            pl.BlockSpec((N, HID, HD), lambda i: (0, 0, 0)),
        ],
        out_specs=pl.BlockSpec((TM, N * HD), lambda i: (i, 0)),
        out_shape=jax.ShapeDtypeStruct((T, N * HD), jnp.float32),
        scratch_shapes=[
            pltpu.VMEM((D, N * HID), jnp.bfloat16),
            pltpu.VMEM((N // PACK, KP, KP), jnp.bfloat16),
            pltpu.VMEM((N, N * HD), jnp.bfloat16),
        ],
        compiler_params=pltpu.CompilerParams(dimension_semantics=("arbitrary",)),
    )(temperature.reshape(1, 1), x.reshape(T, D), router_w, head_centroids,
      input_experts, output_experts)
    return out.reshape(B, S, N * HD)


# speedup vs baseline: 1.6927x; 1.2014x over previous
import math

import jax
import jax.numpy as jnp
from jax import lax
from jax.experimental import pallas as pl
from jax.experimental.pallas import tpu as pltpu

B, S, D = 2, 2048, 1024
N, HID, HD, G = 16, 64, 64, 64
T = B * S
TM = 1024
SB = S // TM
PACK = 4
KP = PACK * HID

_TT = (((1,), (1,)), ((), ()))
_NT = (((1,), (0,)), ((), ()))
_TN = (((0,), (0,)), ((), ()))


def _gelu_tanh(h):
    u = h * (jnp.float32(0.7978845608028654)
             + jnp.float32(0.7978845608028654 * 0.044715) * (h * h))
    r = 0.5 * h
    return r + r * jnp.tanh(u)


def _body(temp_ref, x_ref, rw_ref, c_ref, win_ref, wout_ref, out_ref,
          win2_s, bd_s, emat_s):
    @pl.when(pl.program_id(0) == 0)
    def _prep():
        for n in range(N):
            win2_s[:, n * HID:(n + 1) * HID] = win_ref[n].astype(jnp.bfloat16)
        bd_s[:] = jnp.zeros((N // PACK, KP, KP), jnp.bfloat16)
        for g in range(N // PACK):
            for a in range(PACK):
                bd_s[g, a * HID:(a + 1) * HID, a * HD:(a + 1) * HD] = (
                    wout_ref[g * PACK + a].astype(jnp.bfloat16))
        lane = lax.broadcasted_iota(jnp.int32, (N, N * HD), 1) // HD
        sub = lax.broadcasted_iota(jnp.int32, (N, N * HD), 0)
        emat_s[:] = (lane == sub).astype(jnp.bfloat16)

    xb = x_ref[0].astype(jnp.bfloat16)

    zt = lax.dot_general(rw_ref[:].astype(jnp.bfloat16), xb, _TT,
                         preferred_element_type=jnp.float32)
    znt = zt / jnp.maximum(jnp.sqrt(jnp.sum(zt * zt, axis=0, keepdims=True)), 1e-12)
    c = c_ref[:]
    cn = c / jnp.maximum(jnp.sqrt(jnp.sum(c * c, axis=1, keepdims=True)), 1e-12)
    lt = lax.dot_general(cn.astype(jnp.bfloat16), znt.astype(jnp.bfloat16),
                         _NT, preferred_element_type=jnp.float32)
    lt = lt * (jnp.exp(temp_ref[0]) / math.sqrt(G))

    nidx = lax.broadcasted_iota(jnp.int32, (N, TM), 0)
    neg = jnp.float32(-1e30)
    m1 = jnp.max(lt, axis=0, keepdims=True)
    i1 = jnp.min(jnp.where(lt == m1, nidx, N), axis=0, keepdims=True)
    l2 = jnp.where(nidx == i1, neg, lt)
    m2 = jnp.max(l2, axis=0, keepdims=True)
    i2 = jnp.min(jnp.where(l2 == m2, nidx, N), axis=0, keepdims=True)

    e = jnp.exp(lt - m1)
    probs = e / jnp.sum(e, axis=0, keepdims=True)
    p1 = jnp.max(probs, axis=0, keepdims=True)
    p2 = jnp.max(jnp.where(nidx == i1, neg, probs), axis=0, keepdims=True)
    wt = jnp.where(nidx == i1, p1, 0.0) + jnp.where(nidx == i2, p2, 0.0)
    wexp = lax.dot_general(wt.astype(jnp.bfloat16), emat_s[:], _TN,
                           preferred_element_type=jnp.float32)

    hidden = jnp.dot(xb, win2_s[:], preferred_element_type=jnp.float32)
    hb = _gelu_tanh(hidden).astype(jnp.bfloat16)
    for g in range(N // PACK):
        out_ref[0, :, g * KP:(g + 1) * KP] = lax.dot_general(
            hb[:, g * KP:(g + 1) * KP], bd_s[g], _NT,
            preferred_element_type=jnp.float32) * wexp[:, g * KP:(g + 1) * KP]


def kernel(x, router_w, head_centroids, temperature, input_experts, output_experts):
    return pl.pallas_call(
        _body,
        grid=(T // TM,),
        in_specs=[
            pl.BlockSpec(memory_space=pltpu.SMEM),
            pl.BlockSpec((1, TM, D), lambda i: (i // SB, i % SB, 0)),
            pl.BlockSpec((G, D), lambda i: (0, 0)),
            pl.BlockSpec((N, G), lambda i: (0, 0)),
            pl.BlockSpec((N, D, HID), lambda i: (0, 0, 0)),
            pl.BlockSpec((N, HID, HD), lambda i: (0, 0, 0)),
        ],
        out_specs=pl.BlockSpec((1, TM, N * HD), lambda i: (i // SB, i % SB, 0)),
        out_shape=jax.ShapeDtypeStruct((B, S, N * HD), jnp.float32),
        scratch_shapes=[
            pltpu.VMEM((D, N * HID), jnp.bfloat16),
            pltpu.VMEM((N // PACK, KP, KP), jnp.bfloat16),
            pltpu.VMEM((N, N * HD), jnp.bfloat16),
        ],
        compiler_params=pltpu.CompilerParams(dimension_semantics=("arbitrary",)),
    )(temperature, x, router_w, head_centroids, input_experts, output_experts)
